# baseline (speedup 1.0000x reference)
import jax
import jax.numpy as jnp
from jax import lax
from jax.experimental import pallas as pl
from jax.experimental.pallas import tpu as pltpu
from jax.experimental.pallas import tpu_sc as plsc

NC = 2
NS = 16
NW = NC * NS
CH = 128


def _elu01(t):
    return jnp.where(t > 0, t, 0.1 * (jnp.exp(t) - 1.0))


def _make_sc_agg(n, n_acc, epad, feat, t0, t1):
    assert NS * (t0 + t1) * CH == epad
    assert t0 % 4 == 0 and t1 % 4 == 0
    assert t0 >= 8 and (t1 == 0 or t1 >= 8)
    nco = NC if t1 else 1
    rpt = (n // NS) // 8 * 8
    rtail = n - NS * rpt
    zch = n_acc // (NS * CH)

    out_type = [jax.ShapeDtypeStruct((nco, n, feat), jnp.float32)]
    scratch = [
        pltpu.VMEM_SHARED((n_acc, feat), jnp.float32),
        pltpu.VMEM((4, 2, CH), jnp.int32),
        pltpu.VMEM((CH, feat), jnp.float32),
        pltpu.VMEM((CH, feat), jnp.float32),
        pltpu.SemaphoreType.DMA,
        pltpu.SemaphoreType.DMA,
        pltpu.SemaphoreType.DMA,
        pltpu.SemaphoreType.DMA,
        pltpu.SemaphoreType.DMA,
        pltpu.SemaphoreType.DMA,
        pltpu.SemaphoreType.DMA,
        pltpu.SemaphoreType.DMA,
    ]

    zoff = list(range(0, feat - 15, 16))
    if feat % 16:
        zoff.append(feat - 16)

    def body(x_hbm, eidx_hbm, out_hbm, acc, idx, rowsA, rowsB,
             gA, gB, sA, sB, i0, i1, i2, i3):
        isems = (i0, i1, i2, i3)
        c = lax.axis_index("c")
        s = lax.axis_index("s")

        def whole():
            cpw = jnp.where(c == 0, t0, t1) if t1 else t0
            base = (jnp.where(c == 0, s * t0, NS * t0 + s * t1)
                    if t1 else s * t0)

            pltpu.sync_copy(eidx_hbm.at[base], idx.at[0])
            pltpu.sync_copy(eidx_hbm.at[base + 1], idx.at[1])
            pltpu.async_copy(eidx_hbm.at[base + 2], idx.at[2], i2)
            pltpu.async_copy(eidx_hbm.at[base + 3], idx.at[3], i3)

            pltpu.async_copy(x_hbm.at[idx.at[1, 0]], rowsB, gB)

            z16 = jnp.zeros((16,), jnp.float32)

            @pl.loop(0, CH)
            def _(r):
                for j in zoff:
                    rowsA[r, pl.ds(j, 16)] = z16

            zbase = s * (zch * CH)

            @pl.loop(0, zch)
            def _(z):
                pltpu.async_copy(rowsA, acc.at[pl.ds(zbase + z * CH, CH)], sA)

            @pl.loop(0, zch)
            def _(z):
                pltpu.make_async_copy(rowsA, acc.at[pl.ds(zbase, CH)],
                                      sA).wait()

            pltpu.async_copy(x_hbm.at[idx.at[0, 0]], rowsA, gA)

            plsc.subcore_barrier()

            @pl.loop(0, cpw, step=4)
            def _(k):
                for st in range(4):
                    kk = k + st
                    rows, gsem, ssem = ((rowsA, gA, sA) if st % 2 == 0
                                        else (rowsB, gB, sB))
                    pltpu.make_async_copy(
                        x_hbm.at[idx.at[st, 0]], rows, gsem).wait()
                    pltpu.async_copy(rows, acc.at[idx.at[st, 1]], ssem,
                                     add=True)

                    @pl.when(kk + 2 < cpw)
                    def _():
                        pltpu.make_async_copy(
                            rows, acc.at[idx.at[st, 1]], ssem).wait()

                        @pl.when(kk + 4 < cpw)
                        def _():
                            pltpu.async_copy(eidx_hbm.at[base + kk + 4],
                                             idx.at[st], isems[st])
                        pltpu.make_async_copy(
                            eidx_hbm.at[base + kk + 2],
                            idx.at[(st + 2) % 4], isems[(st + 2) % 4]).wait()
                        pltpu.async_copy(
                            x_hbm.at[idx.at[(st + 2) % 4, 0]], rows, gsem)

            pltpu.make_async_copy(rowsA, acc.at[idx.at[2, 1]], sA).wait()
            pltpu.make_async_copy(rowsB, acc.at[idx.at[3, 1]], sB).wait()

            plsc.subcore_barrier()

            rb = s * rpt
            pltpu.sync_copy(acc.at[pl.ds(rb, rpt)],
                            out_hbm.at[c, pl.ds(rb, rpt)])
            if rtail:
                @pl.when(s == NS - 1)
                def _():
                    tb = NS * rpt
                    pltpu.sync_copy(acc.at[pl.ds(tb, rtail)],
                                    out_hbm.at[c, pl.ds(tb, rtail)])

        if t1 == 0:
            pl.when(c == 0)(whole)
        else:
            whole()

    return pl.kernel(
        body,
        out_type=out_type,
        mesh=plsc.VectorSubcoreMesh(core_axis_name="c", subcore_axis_name="s"),
        scratch_types=scratch,
        compiler_params=pltpu.CompilerParams(use_tc_tiling_on_sc=False),
    )


R = 1000


def _tc_recover_body(x, We, be, Wd, bd, xr, xe):
    xv = x[...]
    xc = jnp.dot(xv, We[...], preferred_element_type=jnp.float32)
    xc = xc + be[...][None, :]
    xr[...] = jnp.dot(xc, Wd[...], preferred_element_type=jnp.float32) \
        + bd[...][None, :]
    xe[:, :128] = xv
    xe[:, 128:] = jnp.ones((xv.shape[0], 8), jnp.float32)


def _tc_mid_body(p0, We, W1, W2, b1, b2, be, W3a, W3b, out):
    aggw = p0[0]
    agg = aggw[:, :128]
    deg = aggw[:, 128:129]
    m1 = jnp.dot(We[...], W1[...], preferred_element_type=jnp.float32)
    v1 = jnp.dot(be[...][None, :], W1[...], preferred_element_type=jnp.float32)
    h2 = jnp.dot(agg, W2[...], preferred_element_type=jnp.float32) \
        + b2[...][None, :]
    h1 = jnp.dot(agg, m1, preferred_element_type=jnp.float32) \
        + deg * v1 + b1[...][None, :]
    out[...] = jnp.dot(_elu01(h2), W3a[...], preferred_element_type=jnp.float32) \
        + jnp.dot(_elu01(h1), W3b[...], preferred_element_type=jnp.float32)


def _tc_softmax_body(q0, b3, out):
    x3 = q0[0] + b3[...][None, :]
    m = jnp.max(x3, axis=1, keepdims=True)
    e = x3 - m
    lse = jnp.log(jnp.sum(jnp.exp(e), axis=1, keepdims=True))
    out[...] = e - lse


def _row_spec(f):
    return pl.BlockSpec((R, f), lambda i: (i, 0))


def _part_spec(core, f):
    return pl.BlockSpec((1, R, f), lambda i, _c=core: (_c, i, 0))


def _full_spec(shape):
    nd = len(shape)
    return pl.BlockSpec(shape, (lambda i: (0,) * nd))


def kernel(x_in, edge_index, We, be, Wd, bd, W1, b1, W2, b2, W3, b3):
    n, nfeat = x_in.shape
    e = edge_index.shape[1]
    assert n % NS == 0
    src = edge_index[0]
    dst = edge_index[1]
    epad = -(-e // (NW * CH * 4)) * (NW * CH * 4)
    if epad != e:
        pad = epad - e
        src = jnp.concatenate([src, jnp.zeros((pad,), jnp.int32)])
        dst = jnp.concatenate([dst, jnp.full((pad,), n, jnp.int32)])
    eidx = jnp.stack([src.reshape(epad // CH, CH),
                      dst.reshape(epad // CH, CH)], axis=1)
    n_acc = -(-(n + 1) // (NS * CH)) * (NS * CH)
    nch = epad // (NS * CH)
    grid = n // R
    wext = nfeat + 8
    nclass = W3.shape[1]

    x_recover, x_ext = pl.pallas_call(
        _tc_recover_body,
        grid=(grid,),
        in_specs=[_row_spec(nfeat), _full_spec(We.shape), _full_spec(be.shape),
                  _full_spec(Wd.shape), _full_spec(bd.shape)],
        out_specs=[_row_spec(nfeat), _row_spec(wext)],
        out_shape=[jax.ShapeDtypeStruct((n, nfeat), jnp.float32),
                   jax.ShapeDtypeStruct((n, wext), jnp.float32)],
    )(x_in, We, be, Wd, bd)

    P, = _make_sc_agg(n, n_acc, epad, wext, nch, 0)(x_ext, eidx)

    support3 = pl.pallas_call(
        _tc_mid_body,
        grid=(grid,),
        in_specs=[_part_spec(0, wext),
                  _full_spec(We.shape), _full_spec(W1.shape),
                  _full_spec(W2.shape), _full_spec(b1.shape),
                  _full_spec(b2.shape), _full_spec(be.shape),
                  _full_spec((nfeat, nclass)), _full_spec((nfeat, nclass))],
        out_specs=_row_spec(nclass),
        out_shape=jax.ShapeDtypeStruct((n, nclass), jnp.float32),
    )(P, We, W1, W2, b1, b2, be, W3[:nfeat], W3[nfeat:])

    Q, = _make_sc_agg(n, n_acc, epad, nclass, nch, 0)(support3, eidx)

    x_out = pl.pallas_call(
        _tc_softmax_body,
        grid=(grid,),
        in_specs=[_part_spec(0, nclass), _full_spec(b3.shape)],
        out_specs=_row_spec(nclass),
        out_shape=jax.ShapeDtypeStruct((n, nclass), jnp.float32),
    )(Q, b3)

    return (x_out, x_recover)


# speedup vs baseline: 1.3256x; 1.3256x over previous
import jax
import jax.numpy as jnp
from jax import lax
from jax.experimental import pallas as pl
from jax.experimental.pallas import tpu as pltpu
from jax.experimental.pallas import tpu_sc as plsc

NC = 2
NS = 16
NW = NC * NS
CH = 128


def _elu01(t):
    return jnp.where(t > 0, t, 0.1 * (jnp.exp(t) - 1.0))


def _make_sc_agg(n, n_acc, epad, feat, t0, t1):
    assert NS * (t0 + t1) * CH == epad
    assert t0 % 4 == 0 and t1 % 4 == 0
    assert t0 >= 8 and (t1 == 0 or t1 >= 8)
    nco = NC if t1 else 1
    rpt = (n // NS) // 8 * 8
    rtail = n - NS * rpt
    zch = n_acc // (NS * CH)

    out_type = [jax.ShapeDtypeStruct((nco, n, feat), jnp.float32)]
    scratch = [
        pltpu.VMEM_SHARED((n_acc, feat), jnp.float32),
        pltpu.VMEM((4, 2, CH), jnp.int32),
        pltpu.VMEM((CH, feat), jnp.float32),
        pltpu.VMEM((CH, feat), jnp.float32),
        pltpu.SemaphoreType.DMA,
        pltpu.SemaphoreType.DMA,
        pltpu.SemaphoreType.DMA,
        pltpu.SemaphoreType.DMA,
        pltpu.SemaphoreType.DMA,
        pltpu.SemaphoreType.DMA,
        pltpu.SemaphoreType.DMA,
        pltpu.SemaphoreType.DMA,
    ]

    zoff = list(range(0, feat - 15, 16))
    if feat % 16:
        zoff.append(feat - 16)

    def body(x_hbm, eidx_hbm, out_hbm, acc, idx, rowsA, rowsB,
             gA, gB, sA, sB, i0, i1, i2, i3):
        isems = (i0, i1, i2, i3)
        c = lax.axis_index("c")
        s = lax.axis_index("s")

        def whole():
            cpw = jnp.where(c == 0, t0, t1) if t1 else t0
            base = (jnp.where(c == 0, s * t0, NS * t0 + s * t1)
                    if t1 else s * t0)

            pltpu.sync_copy(eidx_hbm.at[base], idx.at[0])
            pltpu.sync_copy(eidx_hbm.at[base + 1], idx.at[1])
            pltpu.async_copy(eidx_hbm.at[base + 2], idx.at[2], i2)
            pltpu.async_copy(eidx_hbm.at[base + 3], idx.at[3], i3)

            pltpu.async_copy(x_hbm.at[idx.at[1, 0]], rowsB, gB)

            z16 = jnp.zeros((16,), jnp.float32)

            @pl.loop(0, CH)
            def _(r):
                for j in zoff:
                    rowsA[r, pl.ds(j, 16)] = z16

            zbase = s * (zch * CH)

            @pl.loop(0, zch)
            def _(z):
                pltpu.async_copy(rowsA, acc.at[pl.ds(zbase + z * CH, CH)], sA)

            @pl.loop(0, zch)
            def _(z):
                pltpu.make_async_copy(rowsA, acc.at[pl.ds(zbase, CH)],
                                      sA).wait()

            pltpu.async_copy(x_hbm.at[idx.at[0, 0]], rowsA, gA)

            plsc.subcore_barrier()

            @pl.loop(0, cpw, step=4)
            def _(k):
                for st in range(4):
                    kk = k + st
                    rows, gsem, ssem = ((rowsA, gA, sA) if st % 2 == 0
                                        else (rowsB, gB, sB))
                    pltpu.make_async_copy(
                        x_hbm.at[idx.at[st, 0]], rows, gsem).wait()
                    pltpu.async_copy(rows, acc.at[idx.at[st, 1]], ssem,
                                     add=True)

                    @pl.when(kk + 2 < cpw)
                    def _():
                        pltpu.make_async_copy(
                            rows, acc.at[idx.at[st, 1]], ssem).wait()

                        @pl.when(kk + 4 < cpw)
                        def _():
                            pltpu.async_copy(eidx_hbm.at[base + kk + 4],
                                             idx.at[st], isems[st])
                        pltpu.make_async_copy(
                            eidx_hbm.at[base + kk + 2],
                            idx.at[(st + 2) % 4], isems[(st + 2) % 4]).wait()
                        pltpu.async_copy(
                            x_hbm.at[idx.at[(st + 2) % 4, 0]], rows, gsem)

            pltpu.make_async_copy(rowsA, acc.at[idx.at[2, 1]], sA).wait()
            pltpu.make_async_copy(rowsB, acc.at[idx.at[3, 1]], sB).wait()

            plsc.subcore_barrier()

            rb = s * rpt
            pltpu.sync_copy(acc.at[pl.ds(rb, rpt)],
                            out_hbm.at[c, pl.ds(rb, rpt)])
            if rtail:
                @pl.when(s == NS - 1)
                def _():
                    tb = NS * rpt
                    pltpu.sync_copy(acc.at[pl.ds(tb, rtail)],
                                    out_hbm.at[c, pl.ds(tb, rtail)])

        if t1 == 0:
            pl.when(c == 0)(whole)
        else:
            whole()

    return pl.kernel(
        body,
        out_type=out_type,
        mesh=plsc.VectorSubcoreMesh(core_axis_name="c", subcore_axis_name="s"),
        scratch_types=scratch,
        compiler_params=pltpu.CompilerParams(use_tc_tiling_on_sc=False),
    )


R = 1000


def _tc_recover_body(x, We, be, Wd, bd, xr, xe):
    xv = x[...]
    xc = jnp.dot(xv, We[...], preferred_element_type=jnp.float32)
    xc = xc + be[...][None, :]
    xr[...] = jnp.dot(xc, Wd[...], preferred_element_type=jnp.float32) \
        + bd[...][None, :]
    xe[:, :128] = xv
    xe[:, 128:] = jnp.ones((xv.shape[0], 8), jnp.float32)


def _tc_mid_body(p0, p1, We, W1, W2, b1, b2, be, W3a, W3b, out):
    aggw = p0[0] + p1[0]
    agg = aggw[:, :128]
    deg = aggw[:, 128:129]
    m1 = jnp.dot(We[...], W1[...], preferred_element_type=jnp.float32)
    v1 = jnp.dot(be[...][None, :], W1[...], preferred_element_type=jnp.float32)
    h2 = jnp.dot(agg, W2[...], preferred_element_type=jnp.float32) \
        + b2[...][None, :]
    h1 = jnp.dot(agg, m1, preferred_element_type=jnp.float32) \
        + deg * v1 + b1[...][None, :]
    out[...] = jnp.dot(_elu01(h2), W3a[...], preferred_element_type=jnp.float32) \
        + jnp.dot(_elu01(h1), W3b[...], preferred_element_type=jnp.float32)


def _tc_softmax_body(q0, q1, b3, out):
    x3 = q0[0] + q1[0] + b3[...][None, :]
    m = jnp.max(x3, axis=1, keepdims=True)
    e = x3 - m
    lse = jnp.log(jnp.sum(jnp.exp(e), axis=1, keepdims=True))
    out[...] = e - lse


def _row_spec(f):
    return pl.BlockSpec((R, f), lambda i: (i, 0))


def _part_spec(core, f):
    return pl.BlockSpec((1, R, f), lambda i, _c=core: (_c, i, 0))


def _full_spec(shape):
    nd = len(shape)
    return pl.BlockSpec(shape, (lambda i: (0,) * nd))


def kernel(x_in, edge_index, We, be, Wd, bd, W1, b1, W2, b2, W3, b3):
    n, nfeat = x_in.shape
    e = edge_index.shape[1]
    assert n % NS == 0
    src = edge_index[0]
    dst = edge_index[1]
    epad = -(-e // (NW * CH * 4)) * (NW * CH * 4)
    if epad != e:
        pad = epad - e
        src = jnp.concatenate([src, jnp.zeros((pad,), jnp.int32)])
        dst = jnp.concatenate([dst, jnp.full((pad,), n, jnp.int32)])
    eidx = jnp.stack([src.reshape(epad // CH, CH),
                      dst.reshape(epad // CH, CH)], axis=1)
    n_acc = -(-(n + 1) // (NS * CH)) * (NS * CH)
    nch = epad // (NS * CH)
    grid = n // R
    wext = nfeat + 8
    nclass = W3.shape[1]

    x_recover, x_ext = pl.pallas_call(
        _tc_recover_body,
        grid=(grid,),
        in_specs=[_row_spec(nfeat), _full_spec(We.shape), _full_spec(be.shape),
                  _full_spec(Wd.shape), _full_spec(bd.shape)],
        out_specs=[_row_spec(nfeat), _row_spec(wext)],
        out_shape=[jax.ShapeDtypeStruct((n, nfeat), jnp.float32),
                   jax.ShapeDtypeStruct((n, wext), jnp.float32)],
    )(x_in, We, be, Wd, bd)

    P, = _make_sc_agg(n, n_acc, epad, wext, nch - 8, 8)(x_ext, eidx)

    support3 = pl.pallas_call(
        _tc_mid_body,
        grid=(grid,),
        in_specs=[_part_spec(0, wext), _part_spec(1, wext),
                  _full_spec(We.shape), _full_spec(W1.shape),
                  _full_spec(W2.shape), _full_spec(b1.shape),
                  _full_spec(b2.shape), _full_spec(be.shape),
                  _full_spec((nfeat, nclass)), _full_spec((nfeat, nclass))],
        out_specs=_row_spec(nclass),
        out_shape=jax.ShapeDtypeStruct((n, nclass), jnp.float32),
    )(P, P, We, W1, W2, b1, b2, be, W3[:nfeat], W3[nfeat:])

    Q, = _make_sc_agg(n, n_acc, epad, nclass, nch - 8, 8)(support3, eidx)

    x_out = pl.pallas_call(
        _tc_softmax_body,
        grid=(grid,),
        in_specs=[_part_spec(0, nclass), _part_spec(1, nclass), _full_spec(b3.shape)],
        out_specs=_row_spec(nclass),
        out_shape=jax.ShapeDtypeStruct((n, nclass), jnp.float32),
    )(Q, Q, b3)

    return (x_out, x_recover)
